# baseline (speedup 1.0000x reference)
import functools

import jax
import jax.numpy as jnp
from jax import lax
from jax.experimental import pallas as pl
from jax.experimental.pallas import tpu as pltpu
from jax.experimental.pallas import tpu_sc as plsc

N = 10000
E = 320000
H = 128

_NC = 2
_NS = 16
_NW = _NC * _NS
_EPT = E // _NW


_NR = 80
_NL = 128
_NP = _NR * _NL
_CHUNK = 16


_WIN = 10112


@functools.lru_cache(maxsize=1)
def _make_sc_segment():
    @functools.partial(
        pl.kernel,
        out_type=[
            jax.ShapeDtypeStruct((_NC, _NR, _NL), jnp.float32),
            jax.ShapeDtypeStruct((_NC, _NR, _NL), jnp.float32),
        ],
        mesh=plsc.VectorSubcoreMesh(core_axis_name="c", subcore_axis_name="s"),
        compiler_params=pltpu.CompilerParams(
            needs_layout_passes=False, use_tc_tiling_on_sc=False),
        scratch_types=[
            pltpu.VMEM((N,), jnp.float32),
            pltpu.VMEM((2, _WIN), jnp.int32),
            pltpu.VMEM((_NR, _NL), jnp.float32),
            pltpu.VMEM((_NR, _NL), jnp.float32),
            pltpu.VMEM((_NR, _NL), jnp.float32),
            pltpu.VMEM((_NR, _NL), jnp.float32),
            pltpu.VMEM((_NR,), jnp.int32),
            pltpu.VMEM_SHARED((_NR, _NL), jnp.float32),
            pltpu.VMEM_SHARED((_NR, _NL), jnp.float32),
        ],
    )
    def _sc_segment(x_hbm, ei_hbm, zeros_hbm, iota_hbm, s_out, d_out,
                    x_v, ei_v, acc_s0, acc_s1, acc_d0, acc_d1, rows_v,
                    sh_s, sh_d):
        c = lax.axis_index("c")
        s = lax.axis_index("s")
        base = (c * _NS + s) * _EPT
        off0 = lax.rem(base, _NL)
        awin0 = base - off0

        pltpu.sync_copy(x_hbm, x_v)
        pltpu.sync_copy(ei_hbm.at[:, pl.ds(awin0, _WIN)], ei_v)
        pltpu.sync_copy(zeros_hbm, acc_s0)
        pltpu.sync_copy(zeros_hbm, acc_s1)
        pltpu.sync_copy(zeros_hbm, acc_d0)
        pltpu.sync_copy(zeros_hbm, acc_d1)
        pltpu.sync_copy(iota_hbm, rows_v)

        @pl.when(s == 0)
        def _():
            pltpu.sync_copy(zeros_hbm, sh_s)
            pltpu.sync_copy(zeros_hbm, sh_d)

        ones16 = jnp.ones((_CHUNK,), jnp.float32)
        zeros16i = jnp.zeros((_CHUNK,), jnp.int32)
        ones16i = jnp.ones((_CHUNK,), jnp.int32)
        iota16 = lax.iota(jnp.int32, _CHUNK)

        _UB = 4
        _SACC = (acc_s0, acc_s1)
        _DACC = (acc_d0, acc_d1)

        @plsc.parallel_loop(0, (_EPT // _CHUNK) // _UB, 1, unroll=4)
        def body(j):
            staged = []
            for k in range(_UB):
                lane = off0 + (j * _UB + k) * _CHUNK + iota16
                sidx = plsc.load_gather(ei_v, [zeros16i, lane])
                didx = plsc.load_gather(ei_v, [ones16i, lane])
                vals = plsc.load_gather(x_v, [sidx])
                staged.append((lax.shift_right_logical(didx, 7),
                               lax.bitwise_and(didx, 127), vals))
            for k, (row, col, vals) in enumerate(staged):
                plsc.addupdate_scatter(_SACC[k % 2], [row, col], vals)
            for k, (row, col, _) in enumerate(staged):
                plsc.addupdate_scatter(_DACC[k % 2], [row, col], ones16)

        for jj in range((_EPT // _CHUNK) // _UB * _UB, _EPT // _CHUNK):
            lane = off0 + jj * _CHUNK + iota16
            sidx = plsc.load_gather(ei_v, [zeros16i, lane])
            didx = plsc.load_gather(ei_v, [ones16i, lane])
            vals = plsc.load_gather(x_v, [sidx])
            row = lax.shift_right_logical(didx, 7)
            col = lax.bitwise_and(didx, 127)
            plsc.addupdate_scatter(acc_s0, [row, col], vals)
            plsc.addupdate_scatter(acc_d0, [row, col], ones16)

        plsc.subcore_barrier()
        pltpu.sync_copy(acc_s0, sh_s.at[rows_v], add=True)
        pltpu.sync_copy(acc_s1, sh_s.at[rows_v], add=True)
        pltpu.sync_copy(acc_d0, sh_d.at[rows_v], add=True)
        pltpu.sync_copy(acc_d1, sh_d.at[rows_v], add=True)
        plsc.subcore_barrier()

        @pl.when(s == 0)
        def _():
            pltpu.sync_copy(sh_s, s_out.at[c])
            pltpu.sync_copy(sh_d, d_out.at[c])

    return _sc_segment


_TROWS = 16
_TBLK = _TROWS * _NL


def _tc_body(x_ref, s_ref, d_ref, wenc_ref, benc_ref, wself_ref, wneigh_ref,
             bproc_ref, wdec_ref, bdec_ref, wterm_ref, bterm_ref,
             y_ref, h_ref, t_ref):
    w0 = wenc_ref[0:1, :]
    benc = benc_ref[...]
    u = jnp.dot(w0, wself_ref[...], preferred_element_type=jnp.float32)
    v = jnp.dot(w0, wneigh_ref[...], preferred_element_type=jnp.float32)
    cbias = (jnp.dot(benc, wself_ref[...], preferred_element_type=jnp.float32)
             + bproc_ref[...])
    cflag = jnp.dot(benc, wneigh_ref[...], preferred_element_type=jnp.float32)
    wd = wdec_ref[...]
    wd_h = wd[:H, :]
    alpha = jnp.dot(w0, wd[H:, :], preferred_element_type=jnp.float32)[0, 0]
    gamma = (jnp.dot(benc, wd[H:, :], preferred_element_type=jnp.float32)[0, 0]
             + bdec_ref[0, 0])

    ssum = s_ref[0] + s_ref[1]
    dsum = d_ref[0] + d_ref[1]
    m_rows = ssum / jnp.maximum(dsum, 1.0)
    f_rows = (dsum > 0.0).astype(jnp.float32)
    mT = jnp.transpose(m_rows)
    fT = jnp.transpose(f_rows)
    xT = jnp.transpose(x_ref[...])

    sub_iota = lax.broadcasted_iota(jnp.int32, (_NL, 1), 0)

    bmax = None
    bsum = None
    _FULL = N // _NL
    for r in range(_FULL + 1):
        nrow = min(_NL, N - r * _NL)
        m_col = mT[:, r:r + 1]
        f_col = fT[:, r:r + 1]
        x_col = xT[:, r:r + 1]
        tile = jnp.maximum(
            x_col * u + m_col * v + f_col * cflag + cbias, 0.0)
        h_ref[r * _NL:r * _NL + nrow, :] = tile[:nrow, :]
        yl = (jnp.dot(tile, wd_h, preferred_element_type=jnp.float32)
              + x_col * alpha + gamma)
        y_ref[r * _NL:r * _NL + nrow, :] = jax.nn.sigmoid(yl)[:nrow, :]
        if nrow == _NL:
            tile0 = tile
        else:
            tile0 = jnp.where(sub_iota < nrow, tile, 0.0)
        tmax = jnp.max(tile0, axis=0, keepdims=True)
        tsum = jnp.sum(tile0, axis=0, keepdims=True)
        bmax = tmax if bmax is None else jnp.maximum(bmax, tmax)
        bsum = tsum if bsum is None else bsum + tsum

    wt = wterm_ref[...]
    tv = (jnp.dot(bmax, wt[:H, :], preferred_element_type=jnp.float32)
          + jnp.dot(bsum / N, wt[H:, :], preferred_element_type=jnp.float32))
    t_ref[...] = jax.nn.sigmoid(tv + bterm_ref[...])


def _tc_call(x, s2, d2, W_enc, b_enc, W_self, W_neigh, b_proc,
             W_dec, b_dec, W_term, b_term):
    return pl.pallas_call(
        _tc_body,
        out_shape=[
            jax.ShapeDtypeStruct((N, 1), jnp.float32),
            jax.ShapeDtypeStruct((N, H), jnp.float32),
            jax.ShapeDtypeStruct((1, 1), jnp.float32),
        ],
    )(x, s2, d2, W_enc, b_enc, W_self, W_neigh, b_proc,
      W_dec, b_dec, W_term, b_term)


def kernel(x, edge_index, W_enc, b_enc, W_self, W_neigh, b_proc,
           W_dec, b_dec, W_term, b_term):
    zeros_np = jnp.zeros((_NR, _NL), jnp.float32)
    iota_nr = jnp.arange(_NR, dtype=jnp.int32)
    x_flat = x.reshape(N)
    s2, d2 = _make_sc_segment()(x_flat, edge_index, zeros_np, iota_nr)
    x_grid = jnp.pad(x_flat, (0, _NP - N)).reshape(_NR, _NL)
    y, h, t = _tc_call(
        x_grid, s2, d2, W_enc,
        b_enc.reshape(1, H), W_self, W_neigh, b_proc.reshape(1, H),
        W_dec, b_dec.reshape(1, 1), W_term, b_term.reshape(1, 1))
    return y, h, t.reshape(1)


# speedup vs baseline: 1.0956x; 1.0956x over previous
import functools

import jax
import jax.numpy as jnp
from jax import lax
from jax.experimental import pallas as pl
from jax.experimental.pallas import tpu as pltpu
from jax.experimental.pallas import tpu_sc as plsc

N = 10000
E = 320000
H = 128

_NC = 2
_NS = 16
_NW = _NC * _NS
_EPT = E // _NW


_NR = 80
_NL = 128
_NP = _NR * _NL
_CHUNK = 16


_WIN = 10112


@functools.lru_cache(maxsize=1)
def _make_sc_segment():
    @functools.partial(
        pl.kernel,
        out_type=[
            jax.ShapeDtypeStruct((_NC, _NR, _NL), jnp.float32),
            jax.ShapeDtypeStruct((_NC, _NR, _NL), jnp.float32),
        ],
        mesh=plsc.VectorSubcoreMesh(core_axis_name="c", subcore_axis_name="s"),
        compiler_params=pltpu.CompilerParams(
            needs_layout_passes=False, use_tc_tiling_on_sc=False),
        scratch_types=[
            pltpu.VMEM((N,), jnp.float32),
            pltpu.VMEM((2, _WIN), jnp.int32),
            pltpu.VMEM((_NR, _NL), jnp.float32),
            pltpu.VMEM((_NR, _NL), jnp.float32),
            pltpu.VMEM((_NR,), jnp.int32),
            pltpu.VMEM_SHARED((_NR, _NL), jnp.float32),
            pltpu.VMEM_SHARED((_NR, _NL), jnp.float32),
        ],
    )
    def _sc_segment(x_hbm, ei_hbm, zeros_hbm, s_out, d_out,
                    x_v, ei_v, acc_s, acc_d, rows_v, sh_s, sh_d):
        c = lax.axis_index("c")
        s = lax.axis_index("s")
        base = (c * _NS + s) * _EPT
        off0 = lax.rem(base, _NL)
        awin0 = base - off0

        pltpu.sync_copy(x_hbm, x_v)
        pltpu.sync_copy(ei_hbm.at[:, pl.ds(awin0, _WIN)], ei_v)
        pltpu.sync_copy(zeros_hbm, acc_s)
        pltpu.sync_copy(zeros_hbm, acc_d)

        @pl.when(s == 0)
        def _():
            pltpu.sync_copy(zeros_hbm, sh_s)
            pltpu.sync_copy(zeros_hbm, sh_d)

        ones16 = jnp.ones((_CHUNK,), jnp.float32)
        zeros16i = jnp.zeros((_CHUNK,), jnp.int32)
        ones16i = jnp.ones((_CHUNK,), jnp.int32)
        iota16 = lax.iota(jnp.int32, _CHUNK)

        for k in range(_NR // _CHUNK):
            rows_v[pl.ds(k * _CHUNK, _CHUNK)] = iota16 + k * _CHUNK

        @plsc.parallel_loop(0, _EPT // _CHUNK, 1, unroll=8)
        def body(j):
            lane = off0 + j * _CHUNK + iota16
            sidx = plsc.load_gather(ei_v, [zeros16i, lane])
            didx = plsc.load_gather(ei_v, [ones16i, lane])
            row = lax.shift_right_logical(didx, 7)
            col = lax.bitwise_and(didx, 127)
            vals = plsc.load_gather(x_v, [sidx])
            plsc.addupdate_scatter(acc_s, [row, col], vals)
            plsc.addupdate_scatter(acc_d, [row, col], ones16)

        plsc.subcore_barrier()
        pltpu.sync_copy(acc_s, sh_s.at[rows_v], add=True)
        pltpu.sync_copy(acc_d, sh_d.at[rows_v], add=True)
        plsc.subcore_barrier()

        @pl.when(s == 0)
        def _():
            pltpu.sync_copy(sh_s, s_out.at[c])
            pltpu.sync_copy(sh_d, d_out.at[c])

    return _sc_segment


_TROWS = 16
_TBLK = _TROWS * _NL


def _tc_body(x_ref, s_ref, d_ref, wenc_ref, benc_ref, wself_ref, wneigh_ref,
             bproc_ref, wdec_ref, bdec_ref, wterm_ref, bterm_ref,
             y_ref, h_ref, t_ref, cmax_ref, csum_ref):
    i = pl.program_id(0)
    w0 = wenc_ref[0:1, :]
    benc = benc_ref[...]
    u = jnp.dot(w0, wself_ref[...], preferred_element_type=jnp.float32)
    v = jnp.dot(w0, wneigh_ref[...], preferred_element_type=jnp.float32)
    cbias = (jnp.dot(benc, wself_ref[...], preferred_element_type=jnp.float32)
             + bproc_ref[...])
    cflag = jnp.dot(benc, wneigh_ref[...], preferred_element_type=jnp.float32)
    wd = wdec_ref[...]
    wd_h = wd[:H, :]
    alpha = jnp.dot(w0, wd[H:, :], preferred_element_type=jnp.float32)[0, 0]
    gamma = (jnp.dot(benc, wd[H:, :], preferred_element_type=jnp.float32)[0, 0]
             + bdec_ref[0, 0])

    ssum = s_ref[0] + s_ref[1]
    dsum = d_ref[0] + d_ref[1]
    m_rows = ssum / jnp.maximum(dsum, 1.0)
    f_rows = (dsum > 0.0).astype(jnp.float32)
    mT = jnp.transpose(m_rows)
    fT = jnp.transpose(f_rows)
    xT = jnp.transpose(x_ref[...])

    nbase = i * _TBLK
    sub_iota = lax.broadcasted_iota(jnp.int32, (_NL, 1), 0)

    bmax = None
    bsum = None
    for r in range(_TROWS):
        m_col = mT[:, r:r + 1]
        f_col = fT[:, r:r + 1]
        x_col = xT[:, r:r + 1]
        tile = jnp.maximum(
            x_col * u + m_col * v + f_col * cflag + cbias, 0.0)
        h_ref[r * _NL:(r + 1) * _NL, :] = tile
        yl = (jnp.dot(tile, wd_h, preferred_element_type=jnp.float32)
              + x_col * alpha + gamma)
        y_ref[r * _NL:(r + 1) * _NL, :] = jax.nn.sigmoid(yl)
        valid = (sub_iota + (nbase + r * _NL)) < N
        tile0 = jnp.where(valid, tile, 0.0)
        tmax = jnp.max(tile0, axis=0, keepdims=True)
        tsum = jnp.sum(tile0, axis=0, keepdims=True)
        bmax = tmax if bmax is None else jnp.maximum(bmax, tmax)
        bsum = tsum if bsum is None else bsum + tsum

    @pl.when(i == 0)
    def _():
        cmax_ref[...] = bmax
        csum_ref[...] = bsum

    @pl.when(i > 0)
    def _():
        cmax_ref[...] = jnp.maximum(cmax_ref[...], bmax)
        csum_ref[...] = csum_ref[...] + bsum

    @pl.when(i == pl.num_programs(0) - 1)
    def _():
        wt = wterm_ref[...]
        tv = (jnp.dot(cmax_ref[...], wt[:H, :],
                      preferred_element_type=jnp.float32)
              + jnp.dot(csum_ref[...] / N, wt[H:, :],
                        preferred_element_type=jnp.float32))
        t_ref[...] = jax.nn.sigmoid(tv + bterm_ref[...])


def _tc_call(x, s2, d2, W_enc, b_enc, W_self, W_neigh, b_proc,
             W_dec, b_dec, W_term, b_term):
    grid = (-(-N // _TBLK),)
    full = lambda shape: pl.BlockSpec(shape, lambda i: (0,) * len(shape))
    return pl.pallas_call(
        _tc_body,
        grid=grid,
        in_specs=[
            pl.BlockSpec((_TROWS, _NL), lambda i: (i, 0)),
            pl.BlockSpec((_NC, _TROWS, _NL), lambda i: (0, i, 0)),
            pl.BlockSpec((_NC, _TROWS, _NL), lambda i: (0, i, 0)),
            full((H + 1, H)),
            full((1, H)),
            full((H, H)),
            full((H, H)),
            full((1, H)),
            full((2 * H, 1)),
            full((1, 1)),
            full((2 * H, 1)),
            full((1, 1)),
        ],
        out_specs=[
            pl.BlockSpec((_TBLK, 1), lambda i: (i, 0)),
            pl.BlockSpec((_TBLK, H), lambda i: (i, 0)),
            pl.BlockSpec((1, 1), lambda i: (0, 0)),
        ],
        out_shape=[
            jax.ShapeDtypeStruct((N, 1), jnp.float32),
            jax.ShapeDtypeStruct((N, H), jnp.float32),
            jax.ShapeDtypeStruct((1, 1), jnp.float32),
        ],
        scratch_shapes=[
            pltpu.VMEM((1, H), jnp.float32),
            pltpu.VMEM((1, H), jnp.float32),
        ],
    )(x, s2, d2, W_enc, b_enc, W_self, W_neigh, b_proc,
      W_dec, b_dec, W_term, b_term)


def kernel(x, edge_index, W_enc, b_enc, W_self, W_neigh, b_proc,
           W_dec, b_dec, W_term, b_term):
    zeros_np = jnp.zeros((_NR, _NL), jnp.float32)
    x_flat = x.reshape(N)
    s2, d2 = _make_sc_segment()(x_flat, edge_index, zeros_np)
    x_grid = jnp.pad(x_flat, (0, _NP - N)).reshape(_NR, _NL)
    y, h, t = _tc_call(
        x_grid, s2, d2, W_enc,
        b_enc.reshape(1, H), W_self, W_neigh, b_proc.reshape(1, H),
        W_dec, b_dec.reshape(1, 1), W_term, b_term.reshape(1, 1))
    return y, h, t.reshape(1)


# speedup vs baseline: 1.0956x; 1.0001x over previous
import functools

import jax
import jax.numpy as jnp
from jax import lax
from jax.experimental import pallas as pl
from jax.experimental.pallas import tpu as pltpu
from jax.experimental.pallas import tpu_sc as plsc

N = 10000
E = 320000
H = 128

_NC = 2
_NS = 16
_NW = _NC * _NS
_EPT = E // _NW


_NR = 80
_NL = 128
_NP = _NR * _NL
_CHUNK = 16


_WIN = 10112


@functools.lru_cache(maxsize=1)
def _make_sc_segment():
    @functools.partial(
        pl.kernel,
        out_type=[
            jax.ShapeDtypeStruct((_NC, _NR, _NL), jnp.float32),
            jax.ShapeDtypeStruct((_NC, _NR, _NL), jnp.float32),
        ],
        mesh=plsc.VectorSubcoreMesh(core_axis_name="c", subcore_axis_name="s"),
        compiler_params=pltpu.CompilerParams(
            needs_layout_passes=False, use_tc_tiling_on_sc=False),
        scratch_types=[
            pltpu.VMEM((N,), jnp.float32),
            pltpu.VMEM((2, _WIN), jnp.int32),
            pltpu.VMEM((_NR, _NL), jnp.float32),
            pltpu.VMEM((_NR, _NL), jnp.float32),
            pltpu.VMEM((_NR,), jnp.int32),
            pltpu.VMEM_SHARED((_NR, _NL), jnp.float32),
            pltpu.VMEM_SHARED((_NR, _NL), jnp.float32),
        ],
    )
    def _sc_segment(x_hbm, ei_hbm, zeros_hbm, s_out, d_out,
                    x_v, ei_v, acc_s, acc_d, rows_v, sh_s, sh_d):
        c = lax.axis_index("c")
        s = lax.axis_index("s")
        base = (c * _NS + s) * _EPT
        off0 = lax.rem(base, _NL)
        awin0 = base - off0

        pltpu.sync_copy(x_hbm, x_v)
        pltpu.sync_copy(ei_hbm.at[:, pl.ds(awin0, _WIN)], ei_v)
        pltpu.sync_copy(zeros_hbm, acc_s)
        pltpu.sync_copy(zeros_hbm, acc_d)

        @pl.when(s == 0)
        def _():
            pltpu.sync_copy(zeros_hbm, sh_s)
            pltpu.sync_copy(zeros_hbm, sh_d)

        ones16 = jnp.ones((_CHUNK,), jnp.float32)
        zeros16i = jnp.zeros((_CHUNK,), jnp.int32)
        ones16i = jnp.ones((_CHUNK,), jnp.int32)
        iota16 = lax.iota(jnp.int32, _CHUNK)

        for k in range(_NR // _CHUNK):
            rows_v[pl.ds(k * _CHUNK, _CHUNK)] = iota16 + k * _CHUNK

        @plsc.parallel_loop(0, _EPT // _CHUNK, 1, unroll=8)
        def body(j):
            o = off0 + j * _CHUNK
            sidx = ei_v[0, pl.ds(o, _CHUNK)]
            didx = ei_v[1, pl.ds(o, _CHUNK)]
            row = lax.shift_right_logical(didx, 7)
            col = lax.bitwise_and(didx, 127)
            vals = plsc.load_gather(x_v, [sidx])
            plsc.addupdate_scatter(acc_s, [row, col], vals)
            plsc.addupdate_scatter(acc_d, [row, col], ones16)

        plsc.subcore_barrier()
        pltpu.sync_copy(acc_s, sh_s.at[rows_v], add=True)
        pltpu.sync_copy(acc_d, sh_d.at[rows_v], add=True)
        plsc.subcore_barrier()

        @pl.when(s == 0)
        def _():
            pltpu.sync_copy(sh_s, s_out.at[c])
            pltpu.sync_copy(sh_d, d_out.at[c])

    return _sc_segment


_TROWS = 16
_TBLK = _TROWS * _NL


def _tc_body(x_ref, s_ref, d_ref, wenc_ref, benc_ref, wself_ref, wneigh_ref,
             bproc_ref, wdec_ref, bdec_ref, wterm_ref, bterm_ref,
             y_ref, h_ref, t_ref, cmax_ref, csum_ref):
    i = pl.program_id(0)
    w0 = wenc_ref[0:1, :]
    benc = benc_ref[...]
    u = jnp.dot(w0, wself_ref[...], preferred_element_type=jnp.float32)
    v = jnp.dot(w0, wneigh_ref[...], preferred_element_type=jnp.float32)
    cbias = (jnp.dot(benc, wself_ref[...], preferred_element_type=jnp.float32)
             + bproc_ref[...])
    cflag = jnp.dot(benc, wneigh_ref[...], preferred_element_type=jnp.float32)
    wd = wdec_ref[...]
    wd_h = wd[:H, :]
    alpha = jnp.dot(w0, wd[H:, :], preferred_element_type=jnp.float32)[0, 0]
    gamma = (jnp.dot(benc, wd[H:, :], preferred_element_type=jnp.float32)[0, 0]
             + bdec_ref[0, 0])

    ssum = s_ref[0] + s_ref[1]
    dsum = d_ref[0] + d_ref[1]
    m_rows = ssum / jnp.maximum(dsum, 1.0)
    f_rows = (dsum > 0.0).astype(jnp.float32)
    mT = jnp.transpose(m_rows)
    fT = jnp.transpose(f_rows)
    xT = jnp.transpose(x_ref[...])

    nbase = i * _TBLK
    sub_iota = lax.broadcasted_iota(jnp.int32, (_NL, 1), 0)

    bmax = None
    bsum = None
    for r in range(_TROWS):
        m_col = mT[:, r:r + 1]
        f_col = fT[:, r:r + 1]
        x_col = xT[:, r:r + 1]
        tile = jnp.maximum(
            x_col * u + m_col * v + f_col * cflag + cbias, 0.0)
        h_ref[r * _NL:(r + 1) * _NL, :] = tile
        yl = (jnp.dot(tile, wd_h, preferred_element_type=jnp.float32)
              + x_col * alpha + gamma)
        y_ref[r * _NL:(r + 1) * _NL, :] = jax.nn.sigmoid(yl)
        valid = (sub_iota + (nbase + r * _NL)) < N
        tile0 = jnp.where(valid, tile, 0.0)
        tmax = jnp.max(tile0, axis=0, keepdims=True)
        tsum = jnp.sum(tile0, axis=0, keepdims=True)
        bmax = tmax if bmax is None else jnp.maximum(bmax, tmax)
        bsum = tsum if bsum is None else bsum + tsum

    @pl.when(i == 0)
    def _():
        cmax_ref[...] = bmax
        csum_ref[...] = bsum

    @pl.when(i > 0)
    def _():
        cmax_ref[...] = jnp.maximum(cmax_ref[...], bmax)
        csum_ref[...] = csum_ref[...] + bsum

    @pl.when(i == pl.num_programs(0) - 1)
    def _():
        wt = wterm_ref[...]
        tv = (jnp.dot(cmax_ref[...], wt[:H, :],
                      preferred_element_type=jnp.float32)
              + jnp.dot(csum_ref[...] / N, wt[H:, :],
                        preferred_element_type=jnp.float32))
        t_ref[...] = jax.nn.sigmoid(tv + bterm_ref[...])


def _tc_call(x, s2, d2, W_enc, b_enc, W_self, W_neigh, b_proc,
             W_dec, b_dec, W_term, b_term):
    grid = (-(-N // _TBLK),)
    full = lambda shape: pl.BlockSpec(shape, lambda i: (0,) * len(shape))
    return pl.pallas_call(
        _tc_body,
        grid=grid,
        in_specs=[
            pl.BlockSpec((_TROWS, _NL), lambda i: (i, 0)),
            pl.BlockSpec((_NC, _TROWS, _NL), lambda i: (0, i, 0)),
            pl.BlockSpec((_NC, _TROWS, _NL), lambda i: (0, i, 0)),
            full((H + 1, H)),
            full((1, H)),
            full((H, H)),
            full((H, H)),
            full((1, H)),
            full((2 * H, 1)),
            full((1, 1)),
            full((2 * H, 1)),
            full((1, 1)),
        ],
        out_specs=[
            pl.BlockSpec((_TBLK, 1), lambda i: (i, 0)),
            pl.BlockSpec((_TBLK, H), lambda i: (i, 0)),
            pl.BlockSpec((1, 1), lambda i: (0, 0)),
        ],
        out_shape=[
            jax.ShapeDtypeStruct((N, 1), jnp.float32),
            jax.ShapeDtypeStruct((N, H), jnp.float32),
            jax.ShapeDtypeStruct((1, 1), jnp.float32),
        ],
        scratch_shapes=[
            pltpu.VMEM((1, H), jnp.float32),
            pltpu.VMEM((1, H), jnp.float32),
        ],
    )(x, s2, d2, W_enc, b_enc, W_self, W_neigh, b_proc,
      W_dec, b_dec, W_term, b_term)


def kernel(x, edge_index, W_enc, b_enc, W_self, W_neigh, b_proc,
           W_dec, b_dec, W_term, b_term):
    zeros_np = jnp.zeros((_NR, _NL), jnp.float32)
    x_flat = x.reshape(N)
    s2, d2 = _make_sc_segment()(x_flat, edge_index, zeros_np)
    x_grid = jnp.pad(x_flat, (0, _NP - N)).reshape(_NR, _NL)
    y, h, t = _tc_call(
        x_grid, s2, d2, W_enc,
        b_enc.reshape(1, H), W_self, W_neigh, b_proc.reshape(1, H),
        W_dec, b_dec.reshape(1, 1), W_term, b_term.reshape(1, 1))
    return y, h, t.reshape(1)
